# Initial kernel scaffold; baseline (speedup 1.0000x reference)
import jax
import jax.numpy as jnp
from jax.experimental import pallas as pl


def kernel(x, edge_index, W1, b1, W2, b2):
    raise NotImplementedError("write your pallas kernel here")


# speedup vs baseline: 11.4712x; 11.4712x over previous
import functools

import jax
import jax.numpy as jnp
from jax import lax
from jax.experimental import pallas as pl
from jax.experimental.pallas import tpu as pltpu
from jax.experimental.pallas import tpu_sc as plsc

N = 10000
E = 320000
D = 128

NC = 2
NS = 16
NW = NC * NS
CH = 128
K = 79
E_PAD = NW * K * CH
N_PAD = 10240
BLK_PER_SUB = (N_PAD // CH) // NS
PAD_ROW = N
DEG_W = 16

_mesh = plsc.VectorSubcoreMesh(core_axis_name="c", subcore_axis_name="s")
_f32 = jnp.float32


def _deg_body(dst_hbm, out_hbm, dstv, buf, accd):
    c = lax.axis_index("c")
    s = lax.axis_index("s")
    wid = s * NC + c
    pltpu.sync_copy(dst_hbm.at[wid], dstv)

    zero = jnp.zeros((16,), _f32)
    one = jnp.ones((16,), _f32)

    @pl.loop(0, CH)
    def _(r):
        buf[r, :] = zero

    @pl.loop(0, BLK_PER_SUB)
    def _(b):
        off = (s * BLK_PER_SUB + b) * CH
        pltpu.sync_copy(buf, accd.at[pl.ds(off, CH)])

    @pl.loop(0, CH)
    def _(r):
        buf[r, :] = one

    plsc.subcore_barrier()

    @pl.loop(0, K)
    def _(j):
        pltpu.sync_copy(buf, accd.at[dstv.at[j]], add=True)

    plsc.subcore_barrier()

    @pl.loop(0, BLK_PER_SUB)
    def _(b):
        off = (s * BLK_PER_SUB + b) * CH
        pltpu.sync_copy(accd.at[pl.ds(off, CH)], out_hbm.at[c].at[pl.ds(off, CH)])


_deg_call = pl.kernel(
    _deg_body,
    out_type=jax.ShapeDtypeStruct((NC, N_PAD, DEG_W), _f32),
    mesh=_mesh,
    scratch_types=[
        pltpu.VMEM((K, CH), jnp.int32),
        pltpu.VMEM((CH, DEG_W), _f32),
        pltpu.VMEM_SHARED((N_PAD, DEG_W), _f32),
    ],
)


def _seg_body(h_hbm, src_hbm, dst_hbm, out_hbm, srcv, dstv, rows, acc):
    c = lax.axis_index("c")
    s = lax.axis_index("s")
    wid = s * NC + c
    pltpu.sync_copy(src_hbm.at[wid], srcv)
    pltpu.sync_copy(dst_hbm.at[wid], dstv)

    zero = jnp.zeros((16,), _f32)

    @pl.loop(0, CH)
    def _(r):
        @pl.loop(0, D, step=16)
        def _(cc):
            rows[r, pl.ds(cc, 16)] = zero

    @pl.loop(0, BLK_PER_SUB)
    def _(b):
        off = (s * BLK_PER_SUB + b) * CH
        pltpu.sync_copy(rows, acc.at[pl.ds(off, CH)])

    plsc.subcore_barrier()

    @pl.loop(0, K)
    def _(j):
        pltpu.sync_copy(h_hbm.at[srcv.at[j]], rows)
        pltpu.sync_copy(rows, acc.at[dstv.at[j]], add=True)

    plsc.subcore_barrier()

    @pl.loop(0, BLK_PER_SUB)
    def _(b):
        off = (s * BLK_PER_SUB + b) * CH
        pltpu.sync_copy(acc.at[pl.ds(off, CH)], out_hbm.at[c].at[pl.ds(off, CH)])


_seg_call = pl.kernel(
    _seg_body,
    out_type=jax.ShapeDtypeStruct((NC, N_PAD, D), _f32),
    mesh=_mesh,
    scratch_types=[
        pltpu.VMEM((K, CH), jnp.int32),
        pltpu.VMEM((K, CH), jnp.int32),
        pltpu.VMEM((CH, D), _f32),
        pltpu.VMEM_SHARED((N_PAD, D), _f32),
    ],
)


B = 256


def _dinv_block(degp_ref):
    deg = degp_ref[0] + degp_ref[1] + 1.0
    return lax.rsqrt(deg)[:, 0:1]


def _row_mask(i):
    row = lax.broadcasted_iota(jnp.int32, (B, 1), 0) + i * B
    return row < N


def _tc1_body(x_ref, w_ref, degp_ref, out_ref):
    dinv = _dinv_block(degp_ref)
    h = jnp.dot(x_ref[...], w_ref[...], preferred_element_type=_f32)
    out_ref[...] = jnp.where(_row_mask(pl.program_id(0)), h * dinv, 0.0)


_tc1_call = pl.pallas_call(
    _tc1_body,
    grid=(N_PAD // B,),
    in_specs=[
        pl.BlockSpec((B, D), lambda i: (i, 0)),
        pl.BlockSpec((D, D), lambda i: (0, 0)),
        pl.BlockSpec((NC, B, DEG_W), lambda i: (0, i, 0)),
    ],
    out_specs=pl.BlockSpec((B, D), lambda i: (i, 0)),
    out_shape=jax.ShapeDtypeStruct((N_PAD, D), _f32),
)


def _tc2_body(s1_ref, h1_ref, degp_ref, w_ref, b1_ref, out_ref):
    dinv = _dinv_block(degp_ref)
    t = (s1_ref[0] + s1_ref[1] + h1_ref[...]) * dinv + b1_ref[...]
    z = jnp.where(t >= 0, t, 0.01 * t)
    h2 = jnp.dot(z, w_ref[...], preferred_element_type=_f32) * dinv
    out_ref[...] = jnp.where(_row_mask(pl.program_id(0)), h2, 0.0)


_tc2_call = pl.pallas_call(
    _tc2_body,
    grid=(N_PAD // B,),
    in_specs=[
        pl.BlockSpec((NC, B, D), lambda i: (0, i, 0)),
        pl.BlockSpec((B, D), lambda i: (i, 0)),
        pl.BlockSpec((NC, B, DEG_W), lambda i: (0, i, 0)),
        pl.BlockSpec((D, D), lambda i: (0, 0)),
        pl.BlockSpec((1, D), lambda i: (0, 0)),
    ],
    out_specs=pl.BlockSpec((B, D), lambda i: (i, 0)),
    out_shape=jax.ShapeDtypeStruct((N_PAD, D), _f32),
)


def _tc3_body(s2_ref, h2_ref, degp_ref, b2_ref, out_ref):
    dinv = _dinv_block(degp_ref)
    out_ref[...] = (s2_ref[0] + s2_ref[1] + h2_ref[...]) * dinv + b2_ref[...]


_tc3_call = pl.pallas_call(
    _tc3_body,
    grid=(N_PAD // B,),
    in_specs=[
        pl.BlockSpec((NC, B, D), lambda i: (0, i, 0)),
        pl.BlockSpec((B, D), lambda i: (i, 0)),
        pl.BlockSpec((NC, B, DEG_W), lambda i: (0, i, 0)),
        pl.BlockSpec((1, D), lambda i: (0, 0)),
    ],
    out_specs=pl.BlockSpec((B, D), lambda i: (i, 0)),
    out_shape=jax.ShapeDtypeStruct((N_PAD, D), _f32),
)


@jax.jit
def kernel(x, edge_index, W1, b1, W2, b2):
    src = edge_index[0].astype(jnp.int32)
    dst = edge_index[1].astype(jnp.int32)
    pad = jnp.full((E_PAD - E,), PAD_ROW, jnp.int32)
    src_p = jnp.concatenate([src, pad]).reshape(NW, K, CH)
    dst_p = jnp.concatenate([dst, pad]).reshape(NW, K, CH)
    x_pad = jnp.concatenate([x, jnp.zeros((N_PAD - N, D), x.dtype)])

    degp = _deg_call(dst_p)
    h1p = _tc1_call(x_pad, W1, degp)
    s1p = _seg_call(h1p, src_p, dst_p)
    h2p = _tc2_call(s1p, h1p, degp, W2, b1.reshape(1, D))
    s2p = _seg_call(h2p, src_p, dst_p)
    out = _tc3_call(s2p, h2p, degp, b2.reshape(1, D))
    return out[:N]
